# Initial kernel scaffold; baseline (speedup 1.0000x reference)
import jax
import jax.numpy as jnp
from jax.experimental import pallas as pl


def kernel(x, edge_index, edge_attr, batch, W1, b1, W2, b2, W3, b3, fW1, fb1, fW2, fb2, fW3, fb3):
    raise NotImplementedError("write your pallas kernel here")


# speedup vs baseline: 3.4834x; 3.4834x over previous
import jax
import jax.numpy as jnp
from jax import lax
from jax.experimental import pallas as pl
from jax.experimental.pallas import tpu as pltpu
from jax.experimental.pallas import tpu_sc as plsc

N = 50000
E = 800000
G = 64
D = 90
DP = 96
DL = 128
CB = 16
NP6 = 6
NPAD = 51200
STRIPE = NPAD // 16
NC, NS, NW = 2, 16, 32
GW = 128
NGRP = 196
EPW = NGRP * GW
EPAD = EPW * NW
RB = 1024
NBLK = NPAD // RB


def _sc_mesh():
    return plsc.VectorSubcoreMesh(core_axis_name="c", subcore_axis_name="s")


def _deg_body(dstp, wp, zeros, out, dbuf, wbuf, acc):
    c = lax.axis_index("c")
    s = lax.axis_index("s")
    wid = s * NC + c
    pltpu.sync_copy(dstp.at[wid], dbuf)
    pltpu.sync_copy(wp.at[wid], wbuf)
    pltpu.sync_copy(zeros, acc.at[pl.ds(s * STRIPE, STRIPE)])
    plsc.subcore_barrier()

    def grp(g, carry):
        pltpu.sync_copy(wbuf.at[g], acc.at[dbuf.at[g]], add=True)
        return carry

    lax.fori_loop(0, NGRP, grp, 0)
    plsc.subcore_barrier()
    pltpu.sync_copy(acc.at[pl.ds(s * STRIPE, STRIPE)],
                    out.at[c, pl.ds(s * STRIPE, STRIPE)])


def _deg_call(dstp, wp, zeros):
    return pl.kernel(
        _deg_body,
        out_type=jax.ShapeDtypeStruct((NC, NPAD), jnp.float32),
        mesh=_sc_mesh(),
        compiler_params=pltpu.CompilerParams(use_tc_tiling_on_sc=False,
                                             needs_layout_passes=False),
        scratch_types=[
            pltpu.VMEM((NGRP, GW), jnp.int32),
            pltpu.VMEM((NGRP, GW), jnp.float32),
            pltpu.MemorySpace.VMEM_SHARED((NPAD,), jnp.float32),
        ],
    )(dstp, wp, zeros)


def _agg_body(hf, srcp8, dstp, wp, zeros, out, sbuf, dbuf, wbuf, rows, acc, sem):
    c = lax.axis_index("c")
    s = lax.axis_index("s")
    wid = s * NC + c
    pltpu.sync_copy(srcp8.at[wid], sbuf)
    pltpu.sync_copy(dstp.at[wid], dbuf)
    pltpu.sync_copy(wp.at[wid], wbuf)

    lane = lax.iota(jnp.int32, 16)
    for p in range(NP6):
        pltpu.sync_copy(zeros, acc.at[pl.ds(s * STRIPE, STRIPE)])
        plsc.subcore_barrier()

        def grp(g, carry):
            pltpu.async_copy(hf.at[sbuf.at[g]], rows, sem).wait()
            for i in range(GW // 16):
                wv = wbuf[g, pl.ds(i * 16, 16)]
                ridx = lane + i * 16
                for j in range(CB):
                    cidx = jnp.full((16,), j, jnp.int32)
                    v = plsc.load_gather(rows, [ridx, cidx])
                    plsc.store_scatter(rows, [ridx, cidx], v * wv)
            pltpu.sync_copy(rows, acc.at[dbuf.at[g]], add=True)
            return carry

        lax.fori_loop(0, NGRP, grp, 0)
        plsc.subcore_barrier()
        pltpu.sync_copy(acc.at[pl.ds(s * STRIPE, STRIPE)],
                        out.at[c, pl.ds(s * STRIPE, STRIPE), pl.ds(p * CB, CB)])

        if p < NP6 - 1:
            def bump(g, carry):
                for i in range(GW // 16):
                    sl = pl.ds(i * 16, 16)
                    sbuf[g, sl] = sbuf[g, sl] + 1
                return carry

            lax.fori_loop(0, NGRP, bump, 0)


def _agg_call(hf, srcp8, dstp, wp, zeros):
    return pl.kernel(
        _agg_body,
        out_type=jax.ShapeDtypeStruct((NC, NPAD, DL), jnp.float32),
        mesh=_sc_mesh(),
        compiler_params=pltpu.CompilerParams(use_tc_tiling_on_sc=False,
                                             needs_layout_passes=False),
        scratch_types=[
            pltpu.VMEM((NGRP, GW), jnp.int32),
            pltpu.VMEM((NGRP, GW), jnp.int32),
            pltpu.VMEM((NGRP, GW), jnp.float32),
            pltpu.VMEM((GW, CB), jnp.float32),
            pltpu.MemorySpace.VMEM_SHARED((NPAD, CB), jnp.float32),
            pltpu.SemaphoreType.DMA,
        ],
    )(hf, srcp8, dstp, wp, zeros)


def _lrelu(t):
    return jnp.where(t >= 0, t, 0.01 * t)


def _pre_body(x_ref, deg_ref, w_ref, h_ref, dis_ref):
    deg = deg_ref[:, 0:1] + deg_ref[:, 1:2] + 1.0
    dis = lax.rsqrt(deg)
    h = jnp.dot(x_ref[...], w_ref[...], preferred_element_type=jnp.float32)
    h_ref[...] = h * dis
    dis_ref[...] = dis


def _pre_call(xp, deg2t, w1tp):
    return pl.pallas_call(
        _pre_body,
        grid=(NBLK,),
        in_specs=[
            pl.BlockSpec((RB, D), lambda i: (i, 0)),
            pl.BlockSpec((RB, NC), lambda i: (i, 0)),
            pl.BlockSpec((D, DL), lambda i: (0, 0)),
        ],
        out_specs=[
            pl.BlockSpec((RB, DL), lambda i: (i, 0)),
            pl.BlockSpec((RB, 1), lambda i: (i, 0)),
        ],
        out_shape=[
            jax.ShapeDtypeStruct((NPAD, DL), jnp.float32),
            jax.ShapeDtypeStruct((NPAD, 1), jnp.float32),
        ],
    )(xp, deg2t, w1tp)


def _mid_body(acc_ref, h_ref, dis_ref, b_ref, w_ref, o_ref):
    a = acc_ref[0] + acc_ref[1]
    t = a[:, 0:DP] + h_ref[...][:, 0:DP]
    dis = dis_ref[...]
    u = _lrelu(t * dis + b_ref[...])
    o_ref[...] = jnp.dot(u, w_ref[...], preferred_element_type=jnp.float32) * dis


def _mid_call(acc, h, dis, bp, wtp):
    return pl.pallas_call(
        _mid_body,
        grid=(NBLK,),
        in_specs=[
            pl.BlockSpec((NC, RB, DL), lambda i: (0, i, 0)),
            pl.BlockSpec((RB, DL), lambda i: (i, 0)),
            pl.BlockSpec((RB, 1), lambda i: (i, 0)),
            pl.BlockSpec((1, DP), lambda i: (0, 0)),
            pl.BlockSpec((DP, DL), lambda i: (0, 0)),
        ],
        out_specs=pl.BlockSpec((RB, DL), lambda i: (i, 0)),
        out_shape=jax.ShapeDtypeStruct((NPAD, DL), jnp.float32),
    )(acc, h, dis, bp, wtp)


def _post_body(acc_ref, h_ref, dis_ref, b_ref, batch_ref,
               fw1_ref, fb1_ref, fw2_ref, fb2_ref, fw3_ref, fb3_ref,
               out_ref, sums_ref, cnt_ref):
    i = pl.program_id(0)
    a = acc_ref[0] + acc_ref[1]
    t = a[:, 0:DP] + h_ref[...][:, 0:DP]
    x4 = _lrelu(t * dis_ref[...] + b_ref[...])
    gids = lax.broadcasted_iota(jnp.int32, (RB, G), 1)
    oh = (batch_ref[...] == gids).astype(jnp.float32)

    @pl.when(i == 0)
    def _():
        sums_ref[...] = jnp.zeros_like(sums_ref)
        cnt_ref[...] = jnp.zeros_like(cnt_ref)

    sums_ref[...] += lax.dot_general(oh, x4, (((0,), (0,)), ((), ())),
                                     preferred_element_type=jnp.float32)
    cnt_ref[...] += jnp.sum(oh, axis=0)[:, None]

    @pl.when(i == NBLK - 1)
    def _():
        mean = sums_ref[...] / jnp.maximum(cnt_ref[...], 1.0)
        l1 = _lrelu(jnp.dot(mean, fw1_ref[...].T,
                            preferred_element_type=jnp.float32) + fb1_ref[...])
        l2 = _lrelu(jnp.dot(l1, fw2_ref[...].T,
                            preferred_element_type=jnp.float32) + fb2_ref[...])
        out_ref[...] = (jnp.sum(l2 * fw3_ref[...], axis=1, keepdims=True)
                        + fb3_ref[...])


def _post_call(acc, h, dis, bp, batchp, fw1p, fb1p, fw2, fb2p, fw3p, fb3p):
    return pl.pallas_call(
        _post_body,
        grid=(NBLK,),
        in_specs=[
            pl.BlockSpec((NC, RB, DL), lambda i: (0, i, 0)),
            pl.BlockSpec((RB, DL), lambda i: (i, 0)),
            pl.BlockSpec((RB, 1), lambda i: (i, 0)),
            pl.BlockSpec((1, DP), lambda i: (0, 0)),
            pl.BlockSpec((RB, 1), lambda i: (i, 0)),
            pl.BlockSpec((G, DP), lambda i: (0, 0)),
            pl.BlockSpec((1, G), lambda i: (0, 0)),
            pl.BlockSpec((32, G), lambda i: (0, 0)),
            pl.BlockSpec((1, 32), lambda i: (0, 0)),
            pl.BlockSpec((1, 32), lambda i: (0, 0)),
            pl.BlockSpec((1, 1), lambda i: (0, 0)),
        ],
        out_specs=pl.BlockSpec((G, 1), lambda i: (0, 0)),
        out_shape=jax.ShapeDtypeStruct((G, 1), jnp.float32),
        scratch_shapes=[
            pltpu.VMEM((G, DP), jnp.float32),
            pltpu.VMEM((G, 1), jnp.float32),
        ],
    )(acc, h, dis, bp, batchp, fw1p, fb1p, fw2, fb2p, fw3p, fb3p)


def kernel(x, edge_index, edge_attr, batch, W1, b1, W2, b2, W3, b3,
           fW1, fb1, fW2, fb2, fW3, fb3):
    f32 = jnp.float32
    src = edge_index[0].astype(jnp.int32)
    dst = edge_index[1].astype(jnp.int32)
    w = edge_attr.astype(f32)

    npadding = EPAD - E
    fill = (jnp.arange(npadding, dtype=jnp.int32) * 61) % N
    srcp8 = (jnp.concatenate([src, fill]) * 8).reshape(NW, NGRP, GW)
    dstp = jnp.concatenate([dst, fill]).reshape(NW, NGRP, GW)
    wp = jnp.concatenate([w, jnp.zeros((npadding,), f32)]).reshape(NW, NGRP, GW)

    xp = jnp.zeros((NPAD, D), f32).at[:N].set(x)
    batchp = jnp.full((NPAD, 1), -1, jnp.int32).at[:N, 0].set(batch.astype(jnp.int32))

    zeros_deg = jnp.zeros((STRIPE,), f32)
    zeros_blk = jnp.zeros((STRIPE, CB), f32)

    def padw(W):
        wt = jnp.zeros((DP, DL), f32)
        return wt.at[:W.shape[1], :W.shape[0]].set(W.T)

    w1tp = jnp.zeros((D, DL), f32).at[:, :D].set(W1.T)
    w2tp = padw(W2)
    w3tp = padw(W3)
    b1p = jnp.zeros((1, DP), f32).at[0, :D].set(b1)
    b2p = jnp.zeros((1, DP), f32).at[0, :D].set(b2)
    b3p = jnp.zeros((1, DP), f32).at[0, :D].set(b3)
    fw1p = jnp.zeros((G, DP), f32).at[:, :D].set(fW1)
    fb1p = fb1.reshape(1, G)
    fb2p = fb2.reshape(1, 32)
    fw3p = fW3.reshape(1, 32)
    fb3p = fb3.reshape(1, 1)

    deg2 = _deg_call(dstp, wp, zeros_deg)

    h, dis = _pre_call(xp, deg2.T, w1tp)

    acc1 = _agg_call(h.reshape(NPAD * 8, CB), srcp8, dstp, wp, zeros_blk)
    h2 = _mid_call(acc1, h, dis, b1p, w2tp)

    acc2 = _agg_call(h2.reshape(NPAD * 8, CB), srcp8, dstp, wp, zeros_blk)
    h3 = _mid_call(acc2, h2, dis, b2p, w3tp)

    acc3 = _agg_call(h3.reshape(NPAD * 8, CB), srcp8, dstp, wp, zeros_blk)
    out = _post_call(acc3, h3, dis, b3p, batchp,
                     fw1p, fb1p, fW2, fb2p, fw3p, fb3p)
    return out


# speedup vs baseline: 5.7971x; 1.6642x over previous
import jax
import jax.numpy as jnp
from jax import lax
from jax.experimental import pallas as pl
from jax.experimental.pallas import tpu as pltpu
from jax.experimental.pallas import tpu_sc as plsc

N = 50000
E = 800000
G = 64
D = 90
DP = 96
DL = 128
CB = 16
NP6 = 6
NPAD = 51200
STRIPE = NPAD // 16
NC, NS, NW = 2, 16, 32
GW = 128
NGRP = 196
EPW = NGRP * GW
EPAD = EPW * NW
RB = 1024
NBLK = NPAD // RB


def _sc_mesh():
    return plsc.VectorSubcoreMesh(core_axis_name="c", subcore_axis_name="s")


def _deg_body(dstp, wp, zeros, out, dbuf, wbuf, acc):
    c = lax.axis_index("c")
    s = lax.axis_index("s")
    wid = s * NC + c
    pltpu.sync_copy(dstp.at[wid], dbuf)
    pltpu.sync_copy(wp.at[wid], wbuf)
    pltpu.sync_copy(zeros, acc.at[pl.ds(s * STRIPE, STRIPE)])
    plsc.subcore_barrier()

    def grp(g, carry):
        pltpu.sync_copy(wbuf.at[g], acc.at[dbuf.at[g]], add=True)
        return carry

    lax.fori_loop(0, NGRP, grp, 0)
    plsc.subcore_barrier()
    pltpu.sync_copy(acc.at[pl.ds(s * STRIPE, STRIPE)],
                    out.at[c, pl.ds(s * STRIPE, STRIPE)])


def _deg_call(dstp, wp, zeros):
    return pl.kernel(
        _deg_body,
        out_type=jax.ShapeDtypeStruct((NC, NPAD), jnp.float32),
        mesh=_sc_mesh(),
        compiler_params=pltpu.CompilerParams(use_tc_tiling_on_sc=False,
                                             needs_layout_passes=False),
        scratch_types=[
            pltpu.VMEM((NGRP, GW), jnp.int32),
            pltpu.VMEM((NGRP, GW), jnp.float32),
            pltpu.MemorySpace.VMEM_SHARED((NPAD,), jnp.float32),
        ],
    )(dstp, wp, zeros)


def _agg_body(hf, srcp8, dstp, wp, zeros, out, sbuf, dbuf, wbuf,
              r0, r1, acc, g0, g1):
    c = lax.axis_index("c")
    s = lax.axis_index("s")
    wid = s * NC + c
    pltpu.sync_copy(srcp8.at[wid], sbuf)
    pltpu.sync_copy(dstp.at[wid], dbuf)
    pltpu.sync_copy(wp.at[wid], wbuf)

    rows = (r0, r1)
    gsem = (g0, g1)
    lane = lax.iota(jnp.int32, 16)

    def slot(g, b, bn):
        pltpu.make_async_copy(hf.at[sbuf.at[g]], rows[b], gsem[b]).wait()
        gp1 = jnp.minimum(g + 1, NGRP - 1)
        pltpu.async_copy(hf.at[sbuf.at[gp1]], rows[bn], gsem[bn])

        def mul16(i, carry):
            wv = wbuf[g, pl.ds(i * 16, 16)]
            ridx = lane + i * 16
            for j in range(CB):
                cidx = jnp.full((16,), j, jnp.int32)
                v = plsc.load_gather(rows[b], [ridx, cidx])
                plsc.store_scatter(rows[b], [ridx, cidx], v * wv)
            return carry

        lax.fori_loop(0, GW // 16, mul16, 0)
        pltpu.sync_copy(rows[b], acc.at[dbuf.at[g]], add=True)

    for p in range(NP6):
        pltpu.sync_copy(zeros, acc.at[pl.ds(s * STRIPE, STRIPE)])
        plsc.subcore_barrier()

        pltpu.async_copy(hf.at[sbuf.at[0]], rows[0], gsem[0])

        def ring(gg, carry):
            slot(2 * gg, 0, 1)
            slot(2 * gg + 1, 1, 0)
            return carry

        lax.fori_loop(0, NGRP // 2, ring, 0)
        pltpu.make_async_copy(hf.at[sbuf.at[NGRP - 1]], rows[0],
                              gsem[0]).wait()

        plsc.subcore_barrier()
        pltpu.sync_copy(acc.at[pl.ds(s * STRIPE, STRIPE)],
                        out.at[c, pl.ds(s * STRIPE, STRIPE), pl.ds(p * CB, CB)])

        if p < NP6 - 1:
            def bump(g, carry):
                for i in range(GW // 16):
                    sl = pl.ds(i * 16, 16)
                    sbuf[g, sl] = sbuf[g, sl] + 1
                return carry

            lax.fori_loop(0, NGRP, bump, 0)


def _agg_call(hf, srcp8, dstp, wp, zeros):
    return pl.kernel(
        _agg_body,
        out_type=jax.ShapeDtypeStruct((NC, NPAD, DL), jnp.float32),
        mesh=_sc_mesh(),
        compiler_params=pltpu.CompilerParams(use_tc_tiling_on_sc=False,
                                             needs_layout_passes=False),
        scratch_types=[
            pltpu.VMEM((NGRP, GW), jnp.int32),
            pltpu.VMEM((NGRP, GW), jnp.int32),
            pltpu.VMEM((NGRP, GW), jnp.float32),
            pltpu.VMEM((GW, CB), jnp.float32),
            pltpu.VMEM((GW, CB), jnp.float32),
            pltpu.MemorySpace.VMEM_SHARED((NPAD, CB), jnp.float32),
            pltpu.SemaphoreType.DMA,
            pltpu.SemaphoreType.DMA,
        ],
    )(hf, srcp8, dstp, wp, zeros)


def _lrelu(t):
    return jnp.where(t >= 0, t, 0.01 * t)


def _pre_body(x_ref, deg_ref, w_ref, h_ref, dis_ref):
    deg = deg_ref[:, 0:1] + deg_ref[:, 1:2] + 1.0
    dis = lax.rsqrt(deg)
    h = jnp.dot(x_ref[...], w_ref[...], preferred_element_type=jnp.float32)
    h_ref[...] = h * dis
    dis_ref[...] = dis


def _pre_call(xp, deg2t, w1tp):
    return pl.pallas_call(
        _pre_body,
        grid=(NBLK,),
        in_specs=[
            pl.BlockSpec((RB, D), lambda i: (i, 0)),
            pl.BlockSpec((RB, NC), lambda i: (i, 0)),
            pl.BlockSpec((D, DL), lambda i: (0, 0)),
        ],
        out_specs=[
            pl.BlockSpec((RB, DL), lambda i: (i, 0)),
            pl.BlockSpec((RB, 1), lambda i: (i, 0)),
        ],
        out_shape=[
            jax.ShapeDtypeStruct((NPAD, DL), jnp.float32),
            jax.ShapeDtypeStruct((NPAD, 1), jnp.float32),
        ],
    )(xp, deg2t, w1tp)


def _mid_body(acc_ref, h_ref, dis_ref, b_ref, w_ref, o_ref):
    a = acc_ref[0] + acc_ref[1]
    t = a[:, 0:DP] + h_ref[...][:, 0:DP]
    dis = dis_ref[...]
    u = _lrelu(t * dis + b_ref[...])
    o_ref[...] = jnp.dot(u, w_ref[...], preferred_element_type=jnp.float32) * dis


def _mid_call(acc, h, dis, bp, wtp):
    return pl.pallas_call(
        _mid_body,
        grid=(NBLK,),
        in_specs=[
            pl.BlockSpec((NC, RB, DL), lambda i: (0, i, 0)),
            pl.BlockSpec((RB, DL), lambda i: (i, 0)),
            pl.BlockSpec((RB, 1), lambda i: (i, 0)),
            pl.BlockSpec((1, DP), lambda i: (0, 0)),
            pl.BlockSpec((DP, DL), lambda i: (0, 0)),
        ],
        out_specs=pl.BlockSpec((RB, DL), lambda i: (i, 0)),
        out_shape=jax.ShapeDtypeStruct((NPAD, DL), jnp.float32),
    )(acc, h, dis, bp, wtp)


def _post_body(acc_ref, h_ref, dis_ref, b_ref, batch_ref,
               fw1_ref, fb1_ref, fw2_ref, fb2_ref, fw3_ref, fb3_ref,
               out_ref, sums_ref, cnt_ref):
    i = pl.program_id(0)
    a = acc_ref[0] + acc_ref[1]
    t = a[:, 0:DP] + h_ref[...][:, 0:DP]
    x4 = _lrelu(t * dis_ref[...] + b_ref[...])
    gids = lax.broadcasted_iota(jnp.int32, (RB, G), 1)
    oh = (batch_ref[...] == gids).astype(jnp.float32)

    @pl.when(i == 0)
    def _():
        sums_ref[...] = jnp.zeros_like(sums_ref)
        cnt_ref[...] = jnp.zeros_like(cnt_ref)

    sums_ref[...] += lax.dot_general(oh, x4, (((0,), (0,)), ((), ())),
                                     preferred_element_type=jnp.float32)
    cnt_ref[...] += jnp.sum(oh, axis=0)[:, None]

    @pl.when(i == NBLK - 1)
    def _():
        mean = sums_ref[...] / jnp.maximum(cnt_ref[...], 1.0)
        l1 = _lrelu(jnp.dot(mean, fw1_ref[...].T,
                            preferred_element_type=jnp.float32) + fb1_ref[...])
        l2 = _lrelu(jnp.dot(l1, fw2_ref[...].T,
                            preferred_element_type=jnp.float32) + fb2_ref[...])
        out_ref[...] = (jnp.sum(l2 * fw3_ref[...], axis=1, keepdims=True)
                        + fb3_ref[...])


def _post_call(acc, h, dis, bp, batchp, fw1p, fb1p, fw2, fb2p, fw3p, fb3p):
    return pl.pallas_call(
        _post_body,
        grid=(NBLK,),
        in_specs=[
            pl.BlockSpec((NC, RB, DL), lambda i: (0, i, 0)),
            pl.BlockSpec((RB, DL), lambda i: (i, 0)),
            pl.BlockSpec((RB, 1), lambda i: (i, 0)),
            pl.BlockSpec((1, DP), lambda i: (0, 0)),
            pl.BlockSpec((RB, 1), lambda i: (i, 0)),
            pl.BlockSpec((G, DP), lambda i: (0, 0)),
            pl.BlockSpec((1, G), lambda i: (0, 0)),
            pl.BlockSpec((32, G), lambda i: (0, 0)),
            pl.BlockSpec((1, 32), lambda i: (0, 0)),
            pl.BlockSpec((1, 32), lambda i: (0, 0)),
            pl.BlockSpec((1, 1), lambda i: (0, 0)),
        ],
        out_specs=pl.BlockSpec((G, 1), lambda i: (0, 0)),
        out_shape=jax.ShapeDtypeStruct((G, 1), jnp.float32),
        scratch_shapes=[
            pltpu.VMEM((G, DP), jnp.float32),
            pltpu.VMEM((G, 1), jnp.float32),
        ],
    )(acc, h, dis, bp, batchp, fw1p, fb1p, fw2, fb2p, fw3p, fb3p)


def kernel(x, edge_index, edge_attr, batch, W1, b1, W2, b2, W3, b3,
           fW1, fb1, fW2, fb2, fW3, fb3):
    f32 = jnp.float32
    src = edge_index[0].astype(jnp.int32)
    dst = edge_index[1].astype(jnp.int32)
    w = edge_attr.astype(f32)

    npadding = EPAD - E
    fill = (jnp.arange(npadding, dtype=jnp.int32) * 61) % N
    srcp8 = (jnp.concatenate([src, fill]) * 8).reshape(NW, NGRP, GW)
    dstp = jnp.concatenate([dst, fill]).reshape(NW, NGRP, GW)
    wp = jnp.concatenate([w, jnp.zeros((npadding,), f32)]).reshape(NW, NGRP, GW)

    xp = jnp.zeros((NPAD, D), f32).at[:N].set(x)
    batchp = jnp.full((NPAD, 1), -1, jnp.int32).at[:N, 0].set(batch.astype(jnp.int32))

    zeros_deg = jnp.zeros((STRIPE,), f32)
    zeros_blk = jnp.zeros((STRIPE, CB), f32)

    def padw(W):
        wt = jnp.zeros((DP, DL), f32)
        return wt.at[:W.shape[1], :W.shape[0]].set(W.T)

    w1tp = jnp.zeros((D, DL), f32).at[:, :D].set(W1.T)
    w2tp = padw(W2)
    w3tp = padw(W3)
    b1p = jnp.zeros((1, DP), f32).at[0, :D].set(b1)
    b2p = jnp.zeros((1, DP), f32).at[0, :D].set(b2)
    b3p = jnp.zeros((1, DP), f32).at[0, :D].set(b3)
    fw1p = jnp.zeros((G, DP), f32).at[:, :D].set(fW1)
    fb1p = fb1.reshape(1, G)
    fb2p = fb2.reshape(1, 32)
    fw3p = fW3.reshape(1, 32)
    fb3p = fb3.reshape(1, 1)

    deg2 = _deg_call(dstp, wp, zeros_deg)

    h, dis = _pre_call(xp, deg2.T, w1tp)

    acc1 = _agg_call(h.reshape(NPAD * 8, CB), srcp8, dstp, wp, zeros_blk)
    h2 = _mid_call(acc1, h, dis, b1p, w2tp)

    acc2 = _agg_call(h2.reshape(NPAD * 8, CB), srcp8, dstp, wp, zeros_blk)
    h3 = _mid_call(acc2, h2, dis, b2p, w3tp)

    acc3 = _agg_call(h3.reshape(NPAD * 8, CB), srcp8, dstp, wp, zeros_blk)
    out = _post_call(acc3, h3, dis, b3p, batchp,
                     fw1p, fb1p, fW2, fb2p, fw3p, fb3p)
    return out
